# baseline (speedup 1.0000x reference)
import functools

import jax
import jax.numpy as jnp
from jax import lax
from jax.experimental import pallas as pl
from jax.experimental.pallas import tpu as pltpu
from jax.experimental.pallas import tpu_sc as plsc

TOPK = 1000
QUALITY = 0.3
MIN_DIST = 7
BLOCK = 7

B, H, W = 16, 512, 512
HW = H * W
NCAND = 4096
NPAD = 1040
CHUNK = 8192
NCHUNKS = HW // CHUNK


def _conv2d(x, k):
    kh, kw = k.shape
    kern = k.reshape(1, 1, kh, kw).astype(x.dtype)
    return lax.conv_general_dilated(
        x, kern, window_strides=(1, 1), padding='SAME',
        dimension_numbers=('NCHW', 'OIHW', 'NCHW'))


def _response(im_data):
    im = jnp.round(im_data * 255.0)
    sx = jnp.array([[-1., 0., 1.], [-2., 0., 2.], [-1., 0., 1.]],
                   dtype=jnp.float32)
    k2 = jnp.stack([sx, sx.T])[:, None]
    g = lax.conv_general_dilated(im, k2, (1, 1), 'SAME',
                                 dimension_numbers=('NCHW', 'OIHW', 'NCHW'))
    gx = g[:, 0:1]
    gy = g[:, 1:2]
    box = jnp.ones((1, 1, BLOCK, BLOCK), jnp.float32) / float(BLOCK * BLOCK)
    stack = jnp.concatenate([gx * gx, gy * gy, gx * gy], axis=0)
    out = lax.conv_general_dilated(stack.astype(jnp.bfloat16),
                                   box.astype(jnp.bfloat16), (1, 1), 'SAME',
                                   dimension_numbers=('NCHW', 'OIHW', 'NCHW'),
                                   preferred_element_type=jnp.float32)
    Ixx = out[:B]
    Iyy = out[B:2 * B]
    Ixy = out[2 * B:]
    tr = Ixx + Iyy
    dd = jnp.sqrt(jnp.maximum((Ixx - Iyy) ** 2 + 4.0 * Ixy ** 2, 0.0))
    resp = 0.5 * (tr - dd)
    return resp[:, 0]


def _cand_body(r_ref, out_ref):
    x = r_ref[0]
    maxr = jnp.max(x)
    thresh = jnp.float32(QUALITY) * maxr
    ninf = jnp.full((H, MIN_DIST), -jnp.inf, jnp.float32)
    xp = jnp.concatenate([ninf, x, ninf], axis=1)
    m1 = x
    for k in range(2 * MIN_DIST + 1):
        if k == MIN_DIST:
            continue
        m1 = jnp.maximum(m1, xp[:, k:k + W])
    ninf2 = jnp.full((MIN_DIST, W), -jnp.inf, jnp.float32)
    mp = jnp.concatenate([ninf2, m1, ninf2], axis=0)
    pooled = m1
    for k in range(2 * MIN_DIST + 1):
        if k == MIN_DIST:
            continue
        pooled = jnp.maximum(pooled, mp[k:k + H, :])
    keep = jnp.logical_and(x >= thresh, x >= pooled)
    out_ref[0] = jnp.where(keep, x, 0.0)


def _cand_field(r):
    return pl.pallas_call(
        _cand_body,
        grid=(B,),
        in_specs=[pl.BlockSpec((1, H, W), lambda i: (i, 0, 0))],
        out_specs=pl.BlockSpec((1, H, W), lambda i: (i, 0, 0)),
        out_shape=jax.ShapeDtypeStruct((B, H, W), jnp.float32),
    )(r)


_IOTA = None


def _gath16(src, ids):
    return lax.gather(
        src, ids[:, None],
        lax.GatherDimensionNumbers(offset_dims=(), collapsed_slice_dims=(0,),
                                   start_index_map=(0,)),
        (1,), mode=lax.GatherScatterMode.PROMISE_IN_BOUNDS)


def _group_by_digit(d):
    iota = lax.iota(jnp.int32, 16)
    skey = d * 16 + iota
    skey_s, lane_s = plsc.sort_key_val(skey, iota)
    d_s = lax.shift_right_logical(skey_s, 4)
    prev = _gath16(d_s, jnp.maximum(iota - 1, 0))
    startf = jnp.logical_or(d_s != prev, iota == 0)
    spos = plsc.cummax(jnp.where(startf, iota, 0))
    rank = iota - spos
    nxt = _gath16(d_s, jnp.minimum(iota + 1, 15))
    lastf = jnp.logical_or(d_s != nxt, iota == 15)
    return d_s, lane_s, rank, lastf


_sc_mesh = plsc.VectorSubcoreMesh(core_axis_name="c", subcore_axis_name="s")


@functools.partial(
    pl.kernel, mesh=_sc_mesh,
    compiler_params=pltpu.CompilerParams(needs_layout_passes=False),
    out_type=[
        jax.ShapeDtypeStruct((B * HW,), jnp.float32),
        jax.ShapeDtypeStruct((B * TOPK * 4,), jnp.int32),
    ],
    scratch_types=[
        pltpu.VMEM((CHUNK,), jnp.float32),
        pltpu.VMEM((NCAND,), jnp.int32),
        pltpu.VMEM((NCAND,), jnp.int32),
        pltpu.VMEM((NCAND,), jnp.int32),
        pltpu.VMEM((NCAND,), jnp.int32),
        pltpu.VMEM((NPAD,), jnp.int32),
        pltpu.VMEM((32,), jnp.int32),
        pltpu.VMEM((4096,), jnp.int32),
        pltpu.VMEM((1024,), jnp.float32),
        pltpu.VMEM((1, 1024), jnp.int32),
        pltpu.SemaphoreType.DMA,
    ],
)
def _sc_topk(cand_hbm, mask_hbm, kpts_hbm,
             cbuf, k0, i0, k1, i1, padv, obuf, kl, vbuf, fbuf, sem):
    c = lax.axis_index("c")
    s = lax.axis_index("s")
    b = s

    @pl.when(c == 0)
    def _():
        iota = lax.iota(jnp.int32, 16)
        base0 = b * HW

        def zero_body(j, _):
            cbuf[pl.ds(j * 16, 16)] = jnp.zeros((16,), jnp.float32)
            return 0

        lax.fori_loop(0, CHUNK // 16, zero_body, 0)

        def memset_body(t, _):
            pltpu.sync_copy(cbuf, mask_hbm.at[pl.ds(base0 + t * CHUNK, CHUNK)])
            return 0

        lax.fori_loop(0, NCHUNKS, memset_body, 0)

        def chunk_body(t, cur):
            pltpu.sync_copy(cand_hbm.at[pl.ds(base0 + t * CHUNK, CHUNK)], cbuf)

            def vec_body(j, cur):
                v = cbuf[pl.ds(j * 16, 16)]
                m = v > 0.0
                kbits = jnp.bitwise_not(plsc.bitcast(v, jnp.int32))
                idx = t * CHUNK + j * 16 + iota
                plsc.store_compressed(k0.at[pl.ds(cur, 16)], kbits, mask=m)
                plsc.store_compressed(i0.at[pl.ds(cur, 16)], idx, mask=m)
                cnt = jnp.sum(m.astype(jnp.int32))
                return jnp.minimum(cur + cnt, NCAND - 16)

            return lax.fori_loop(0, CHUNK // 16, vec_body, cur)

        cur = lax.fori_loop(0, NCHUNKS, chunk_body, jnp.int32(0))

        k0[pl.ds(cur, 16)] = jnp.full((16,), -1, jnp.int32)
        i0[pl.ds(cur, 16)] = jnp.zeros((16,), jnp.int32)

        pltpu.sync_copy(cand_hbm.at[pl.ds(base0, CHUNK)], cbuf)

        def pad_body(j, pcur):
            v = cbuf[pl.ds(j * 16, 16)]
            m = v == 0.0
            idx = j * 16 + iota
            plsc.store_compressed(padv.at[pl.ds(pcur, 16)], idx, mask=m)
            cnt = jnp.sum(m.astype(jnp.int32))
            return jnp.minimum(pcur + cnt, NPAD - 16)

        lax.fori_loop(0, 128, pad_body, jnp.int32(0))

        nvec = (cur + 15) // 16

        def one_pass(shift, src_k, src_i, dst_k, dst_i):
            obuf[pl.ds(0, 16)] = jnp.zeros((16,), jnp.int32)
            obuf[pl.ds(16, 16)] = jnp.zeros((16,), jnp.int32)

            def hist_body(j, _):
                kv = src_k[pl.ds(j * 16, 16)]
                d = jnp.bitwise_and(lax.shift_right_logical(kv, shift), 31)
                d_s, _, rank, lastf = _group_by_digit(d)
                plsc.addupdate_scatter(obuf.at[:], [d_s], rank + 1, mask=lastf)
                return 0

            lax.fori_loop(0, nvec, hist_body, 0)

            h0 = obuf[pl.ds(0, 16)]
            h1 = obuf[pl.ds(16, 16)]
            e0 = plsc.cumsum(h0) - h0
            e1 = plsc.cumsum(h1) - h1 + jnp.sum(h0)
            obuf[pl.ds(0, 16)] = e0
            obuf[pl.ds(16, 16)] = e1

            def perm_body(j, _):
                kv = src_k[pl.ds(j * 16, 16)]
                iv = src_i[pl.ds(j * 16, 16)]
                d = jnp.bitwise_and(lax.shift_right_logical(kv, shift), 31)
                d_s, lane_s, rank, lastf = _group_by_digit(d)
                basev = plsc.load_gather(obuf.at[:], [d_s])
                pos = basev + rank
                plsc.store_scatter(dst_k.at[:], [pos], _gath16(kv, lane_s))
                plsc.store_scatter(dst_i.at[:], [pos], _gath16(iv, lane_s))
                plsc.addupdate_scatter(obuf.at[:], [d_s], rank + 1, mask=lastf)
                return 0

            lax.fori_loop(0, nvec, perm_body, 0)

        one_pass(0, k0, i0, k1, i1)
        one_pass(5, k1, i1, k0, i0)
        one_pass(10, k0, i0, k1, i1)
        one_pass(15, k1, i1, k0, i0)
        one_pass(20, k0, i0, k1, i1)
        one_pass(25, k1, i1, k0, i0)
        one_pass(30, k0, i0, k1, i1)

        m = jnp.minimum(cur, TOPK)
        for j in range(64):
            lane = j * 16 + iota
            is_c = lane < m
            sidx = i1[pl.ds(j * 16, 16)]
            pidx = plsc.load_gather(padv.at[:], [jnp.maximum(lane - m, 0)])
            fidx = jnp.where(is_c, sidx, pidx)
            vbuf[pl.ds(j * 16, 16)] = jnp.where(
                is_c, jnp.ones((16,), jnp.float32), jnp.zeros((16,), jnp.float32))
            fbuf[0, pl.ds(j * 16, 16)] = fidx + base0
            y = lax.shift_right_logical(fidx, 9)
            x = jnp.bitwise_and(fidx, W - 1)
            zero = jnp.zeros((16,), jnp.int32)
            flat = lane * 4
            plsc.store_scatter(kl.at[:], [flat], jnp.full((16,), b, jnp.int32))
            plsc.store_scatter(kl.at[:], [flat + 1], zero)
            plsc.store_scatter(kl.at[:], [flat + 2], y)
            plsc.store_scatter(kl.at[:], [flat + 3], x)

        pltpu.async_copy(vbuf, mask_hbm.at[fbuf.at[0]], sem).wait()
        pltpu.sync_copy(kl.at[pl.ds(0, TOPK * 4)],
                        kpts_hbm.at[pl.ds(b * TOPK * 4, TOPK * 4)])


def kernel(im_data):
    r = _response(im_data)
    cand = _cand_field(r)
    mask_flat, kpts_flat = _sc_topk(cand.reshape(B * HW))
    return mask_flat.reshape(B, 1, H, W), kpts_flat.reshape(B, TOPK, 4)


# speedup vs baseline: 1.4240x; 1.4240x over previous
import functools

import jax
import jax.numpy as jnp
from jax import lax
from jax.experimental import pallas as pl
from jax.experimental.pallas import tpu as pltpu
from jax.experimental.pallas import tpu_sc as plsc

TOPK = 1000
QUALITY = 0.3
MIN_DIST = 7
BLOCK = 7

B, H, W = 16, 512, 512
HW = H * W
NCAND = 4096
NPAD = 1040
CHUNK = 8192
NCHUNKS = HW // CHUNK


def _conv2d(x, k):
    kh, kw = k.shape
    kern = k.reshape(1, 1, kh, kw).astype(x.dtype)
    return lax.conv_general_dilated(
        x, kern, window_strides=(1, 1), padding='SAME',
        dimension_numbers=('NCHW', 'OIHW', 'NCHW'))


def _response(im_data):
    im = jnp.round(im_data * 255.0)
    sx = jnp.array([[-1., 0., 1.], [-2., 0., 2.], [-1., 0., 1.]],
                   dtype=jnp.float32)
    k2 = jnp.stack([sx, sx.T])[:, None]
    g = lax.conv_general_dilated(im, k2, (1, 1), 'SAME',
                                 dimension_numbers=('NCHW', 'OIHW', 'NCHW'))
    gx = g[:, 0:1]
    gy = g[:, 1:2]
    box = (jnp.ones((1, 1, BLOCK, BLOCK), jnp.float32)
           / float(BLOCK * BLOCK)).astype(jnp.bfloat16)

    def _boxconv(x):
        return lax.conv_general_dilated(
            x.astype(jnp.bfloat16), box, (1, 1), 'SAME',
            dimension_numbers=('NCHW', 'OIHW', 'NCHW'),
            preferred_element_type=jnp.float32)

    Ixx = _boxconv(gx * gx)
    Iyy = _boxconv(gy * gy)
    Ixy = _boxconv(gx * gy)
    tr = Ixx + Iyy
    dd = jnp.sqrt(jnp.maximum((Ixx - Iyy) ** 2 + 4.0 * Ixy ** 2, 0.0))
    resp = 0.5 * (tr - dd)
    return resp[:, 0]


def _cand_body(r_ref, out_ref):
    x = r_ref[0]
    maxr = jnp.max(x)
    thresh = jnp.float32(QUALITY) * maxr
    ninf = jnp.full((H, MIN_DIST), -jnp.inf, jnp.float32)
    xp = jnp.concatenate([ninf, x, ninf], axis=1)
    m1 = x
    for k in range(2 * MIN_DIST + 1):
        if k == MIN_DIST:
            continue
        m1 = jnp.maximum(m1, xp[:, k:k + W])
    ninf2 = jnp.full((MIN_DIST, W), -jnp.inf, jnp.float32)
    mp = jnp.concatenate([ninf2, m1, ninf2], axis=0)
    pooled = m1
    for k in range(2 * MIN_DIST + 1):
        if k == MIN_DIST:
            continue
        pooled = jnp.maximum(pooled, mp[k:k + H, :])
    keep = jnp.logical_and(x >= thresh, x >= pooled)
    out_ref[0] = jnp.where(keep, x, 0.0)


def _cand_field(r):
    return pl.pallas_call(
        _cand_body,
        grid=(B,),
        in_specs=[pl.BlockSpec((1, H, W), lambda i: (i, 0, 0))],
        out_specs=pl.BlockSpec((1, H, W), lambda i: (i, 0, 0)),
        out_shape=jax.ShapeDtypeStruct((B, H, W), jnp.float32),
    )(r)


_IOTA = None


def _gath16(src, ids):
    return lax.gather(
        src, ids[:, None],
        lax.GatherDimensionNumbers(offset_dims=(), collapsed_slice_dims=(0,),
                                   start_index_map=(0,)),
        (1,), mode=lax.GatherScatterMode.PROMISE_IN_BOUNDS)


def _group_by_digit(d):
    iota = lax.iota(jnp.int32, 16)
    skey = d * 16 + iota
    skey_s, lane_s = plsc.sort_key_val(skey, iota)
    d_s = lax.shift_right_logical(skey_s, 4)
    prev = _gath16(d_s, jnp.maximum(iota - 1, 0))
    startf = jnp.logical_or(d_s != prev, iota == 0)
    spos = plsc.cummax(jnp.where(startf, iota, 0))
    rank = iota - spos
    nxt = _gath16(d_s, jnp.minimum(iota + 1, 15))
    lastf = jnp.logical_or(d_s != nxt, iota == 15)
    return d_s, lane_s, rank, lastf


_sc_mesh = plsc.VectorSubcoreMesh(core_axis_name="c", subcore_axis_name="s")


@functools.partial(
    pl.kernel, mesh=_sc_mesh,
    compiler_params=pltpu.CompilerParams(needs_layout_passes=False),
    out_type=[
        jax.ShapeDtypeStruct((B * HW,), jnp.float32),
        jax.ShapeDtypeStruct((B * TOPK * 4,), jnp.int32),
    ],
    scratch_types=[
        pltpu.VMEM((CHUNK,), jnp.float32),
        pltpu.VMEM((NCAND,), jnp.int32),
        pltpu.VMEM((NCAND,), jnp.int32),
        pltpu.VMEM((NCAND,), jnp.int32),
        pltpu.VMEM((NCAND,), jnp.int32),
        pltpu.VMEM((NPAD,), jnp.int32),
        pltpu.VMEM((32,), jnp.int32),
        pltpu.VMEM((4096,), jnp.int32),
        pltpu.VMEM((1024,), jnp.float32),
        pltpu.VMEM((1, 1024), jnp.int32),
        pltpu.SemaphoreType.DMA,
    ],
)
def _sc_topk(cand_hbm, mask_hbm, kpts_hbm,
             cbuf, k0, i0, k1, i1, padv, obuf, kl, vbuf, fbuf, sem):
    c = lax.axis_index("c")
    s = lax.axis_index("s")
    b = s

    @pl.when(c == 0)
    def _():
        iota = lax.iota(jnp.int32, 16)
        base0 = b * HW

        def zero_body(j, _):
            cbuf[pl.ds(j * 16, 16)] = jnp.zeros((16,), jnp.float32)
            return 0

        lax.fori_loop(0, CHUNK // 16, zero_body, 0)

        def memset_body(t, _):
            pltpu.sync_copy(cbuf, mask_hbm.at[pl.ds(base0 + t * CHUNK, CHUNK)])
            return 0

        lax.fori_loop(0, NCHUNKS, memset_body, 0)

        def chunk_body(t, cur):
            pltpu.sync_copy(cand_hbm.at[pl.ds(base0 + t * CHUNK, CHUNK)], cbuf)

            def vec_body(j, cur):
                v = cbuf[pl.ds(j * 16, 16)]
                m = v > 0.0
                kbits = jnp.bitwise_not(plsc.bitcast(v, jnp.int32))
                idx = t * CHUNK + j * 16 + iota
                plsc.store_compressed(k0.at[pl.ds(cur, 16)], kbits, mask=m)
                plsc.store_compressed(i0.at[pl.ds(cur, 16)], idx, mask=m)
                cnt = jnp.sum(m.astype(jnp.int32))
                return jnp.minimum(cur + cnt, NCAND - 16)

            return lax.fori_loop(0, CHUNK // 16, vec_body, cur)

        cur = lax.fori_loop(0, NCHUNKS, chunk_body, jnp.int32(0))

        k0[pl.ds(cur, 16)] = jnp.full((16,), -1, jnp.int32)
        i0[pl.ds(cur, 16)] = jnp.zeros((16,), jnp.int32)

        pltpu.sync_copy(cand_hbm.at[pl.ds(base0, CHUNK)], cbuf)

        def pad_body(j, pcur):
            v = cbuf[pl.ds(j * 16, 16)]
            m = v == 0.0
            idx = j * 16 + iota
            plsc.store_compressed(padv.at[pl.ds(pcur, 16)], idx, mask=m)
            cnt = jnp.sum(m.astype(jnp.int32))
            return jnp.minimum(pcur + cnt, NPAD - 16)

        lax.fori_loop(0, 128, pad_body, jnp.int32(0))

        nvec = (cur + 15) // 16

        def one_pass(shift, src_k, src_i, dst_k, dst_i):
            obuf[pl.ds(0, 16)] = jnp.zeros((16,), jnp.int32)
            obuf[pl.ds(16, 16)] = jnp.zeros((16,), jnp.int32)

            def hist_body(j, _):
                kv = src_k[pl.ds(j * 16, 16)]
                d = jnp.bitwise_and(lax.shift_right_logical(kv, shift), 31)
                d_s, _, rank, lastf = _group_by_digit(d)
                plsc.addupdate_scatter(obuf.at[:], [d_s], rank + 1, mask=lastf)
                return 0

            lax.fori_loop(0, nvec, hist_body, 0)

            h0 = obuf[pl.ds(0, 16)]
            h1 = obuf[pl.ds(16, 16)]
            e0 = plsc.cumsum(h0) - h0
            e1 = plsc.cumsum(h1) - h1 + jnp.sum(h0)
            obuf[pl.ds(0, 16)] = e0
            obuf[pl.ds(16, 16)] = e1

            def perm_body(j, _):
                kv = src_k[pl.ds(j * 16, 16)]
                iv = src_i[pl.ds(j * 16, 16)]
                d = jnp.bitwise_and(lax.shift_right_logical(kv, shift), 31)
                d_s, lane_s, rank, lastf = _group_by_digit(d)
                basev = plsc.load_gather(obuf.at[:], [d_s])
                pos = basev + rank
                plsc.store_scatter(dst_k.at[:], [pos], _gath16(kv, lane_s))
                plsc.store_scatter(dst_i.at[:], [pos], _gath16(iv, lane_s))
                plsc.addupdate_scatter(obuf.at[:], [d_s], rank + 1, mask=lastf)
                return 0

            lax.fori_loop(0, nvec, perm_body, 0)

        one_pass(0, k0, i0, k1, i1)
        one_pass(5, k1, i1, k0, i0)
        one_pass(10, k0, i0, k1, i1)
        one_pass(15, k1, i1, k0, i0)
        one_pass(20, k0, i0, k1, i1)
        one_pass(25, k1, i1, k0, i0)
        one_pass(30, k0, i0, k1, i1)

        m = jnp.minimum(cur, TOPK)
        for j in range(64):
            lane = j * 16 + iota
            is_c = lane < m
            sidx = i1[pl.ds(j * 16, 16)]
            pidx = plsc.load_gather(padv.at[:], [jnp.maximum(lane - m, 0)])
            fidx = jnp.where(is_c, sidx, pidx)
            vbuf[pl.ds(j * 16, 16)] = jnp.where(
                is_c, jnp.ones((16,), jnp.float32), jnp.zeros((16,), jnp.float32))
            fbuf[0, pl.ds(j * 16, 16)] = fidx + base0
            y = lax.shift_right_logical(fidx, 9)
            x = jnp.bitwise_and(fidx, W - 1)
            zero = jnp.zeros((16,), jnp.int32)
            flat = lane * 4
            plsc.store_scatter(kl.at[:], [flat], jnp.full((16,), b, jnp.int32))
            plsc.store_scatter(kl.at[:], [flat + 1], zero)
            plsc.store_scatter(kl.at[:], [flat + 2], y)
            plsc.store_scatter(kl.at[:], [flat + 3], x)

        pltpu.async_copy(vbuf, mask_hbm.at[fbuf.at[0]], sem).wait()
        pltpu.sync_copy(kl.at[pl.ds(0, TOPK * 4)],
                        kpts_hbm.at[pl.ds(b * TOPK * 4, TOPK * 4)])


def kernel(im_data):
    r = _response(im_data)
    cand = _cand_field(r)
    mask_flat, kpts_flat = _sc_topk(cand.reshape(B * HW))
    return mask_flat.reshape(B, 1, H, W), kpts_flat.reshape(B, TOPK, 4)
